# Initial kernel scaffold; baseline (speedup 1.0000x reference)
import jax
import jax.numpy as jnp
from jax.experimental import pallas as pl


def kernel(x, emb):
    raise NotImplementedError("write your pallas kernel here")


# speedup vs baseline: 1.5468x; 1.5468x over previous
import functools

import jax
import jax.numpy as jnp
import numpy as np
from jax import lax
from jax.experimental import pallas as pl
from jax.experimental.pallas import tpu as pltpu
from jax.experimental.pallas import tpu_sc as plsc

F = 26
FIELD_DIM = 3846
FD = F * FIELD_DIM
D = 16
OFFS = np.concatenate([[0], np.cumsum([FIELD_DIM] * F)[:-1]]).astype(np.int32)

NW = 32
GROUP_B = 4
ROWS_B = F * F
ROWS_G = GROUP_B * ROWS_B
CHUNK = 128
NCHUNK = (ROWS_G + CHUNK - 1) // CHUNK
IDX_PAD = NCHUNK * CHUNK


def _sc_gather_ffm(table, idx_blocks, batch):
    n_groups = batch // (NW * GROUP_B)
    mesh = plsc.VectorSubcoreMesh(core_axis_name="c", subcore_axis_name="s")

    @functools.partial(
        pl.kernel,
        out_type=jax.ShapeDtypeStruct((batch, D), jnp.float32),
        mesh=mesh,
        scratch_types=[
            pltpu.VMEM((NCHUNK, CHUNK), jnp.int32),
            pltpu.VMEM((NCHUNK, CHUNK), jnp.int32),
            pltpu.VMEM((IDX_PAD, D), jnp.float32),
            pltpu.VMEM((IDX_PAD, D), jnp.float32),
            pltpu.VMEM((2 * GROUP_B, D), jnp.float32),
            pltpu.SemaphoreType.DMA,
            pltpu.SemaphoreType.DMA,
        ],
        compiler_params=pltpu.CompilerParams(use_tc_tiling_on_sc=False),
    )
    def k(table_h, idx_h, out_h, idxv0, idxv1, rows0, rows1, outb, sem0, sem1):
        wid = lax.axis_index("s") * 2 + lax.axis_index("c")
        gbase = wid * n_groups

        def fire(idxv, rows, sem, blk):
            pltpu.sync_copy(idx_h.at[blk], idxv)
            for j in range(NCHUNK):
                pltpu.make_async_copy(
                    table_h.at[idxv.at[j]],
                    rows.at[pl.ds(j * CHUNK, CHUNK)],
                    sem,
                ).start()

        def drain(idxv, rows, sem):
            for j in range(NCHUNK):
                pltpu.make_async_copy(
                    table_h.at[idxv.at[j]],
                    rows.at[pl.ds(j * CHUNK, CHUNK)],
                    sem,
                ).wait()

        def compute(rows, slot):
            def body_b(bl, c):
                base = bl * ROWS_B
                acc = jnp.zeros((D,), jnp.float32)
                for f in range(F - 1):
                    for g in range(f + 1, F):
                        acc = acc + rows[base + f * F + g, :] * rows[base + g * F + f, :]
                outb[slot * GROUP_B + bl, :] = acc
                return c
            lax.fori_loop(0, GROUP_B, body_b, 0)

        fire(idxv0, rows0, sem0, gbase)

        def step(it, c):
            g0 = 2 * it
            fire(idxv1, rows1, sem1, gbase + g0 + 1)
            drain(idxv0, rows0, sem0)
            compute(rows0, 0)

            @pl.when(g0 + 2 < n_groups)
            def _():
                fire(idxv0, rows0, sem0, gbase + g0 + 2)

            drain(idxv1, rows1, sem1)
            compute(rows1, 1)
            pltpu.sync_copy(
                outb, out_h.at[pl.ds((gbase + g0) * GROUP_B, 2 * GROUP_B)]
            )
            return c

        lax.fori_loop(0, n_groups // 2, step, 0)

    return k(table, idx_blocks)


def _tc_reduce(partials):
    def body(p_ref, o_ref):
        o_ref[...] = jnp.sum(p_ref[...], axis=1)

    return pl.pallas_call(
        body,
        out_shape=jax.ShapeDtypeStruct((partials.shape[0],), jnp.float32),
    )(partials)


def kernel(x, emb):
    batch = x.shape[0]
    offs = jnp.asarray(OFFS, x.dtype)
    xo = x + offs[None, :]
    fofs = (jnp.arange(F, dtype=x.dtype) * FD)[None, :, None]
    idx3 = xo[:, None, :] + fofs
    flat = idx3.reshape(batch // GROUP_B, ROWS_G)
    padded = jnp.pad(flat, ((0, 0), (0, IDX_PAD - ROWS_G)))
    idx_blocks = padded.reshape(batch // GROUP_B, NCHUNK, CHUNK)
    table = emb.reshape(F * FD, D)
    partials = _sc_gather_ffm(table, idx_blocks, batch)
    return _tc_reduce(partials)


# speedup vs baseline: 59.0011x; 38.1445x over previous
import functools

import jax
import jax.numpy as jnp
import numpy as np
from jax import lax
from jax.experimental import pallas as pl
from jax.experimental.pallas import tpu as pltpu
from jax.experimental.pallas import tpu_sc as plsc

F = 26
FIELD_DIM = 3846
FD = F * FIELD_DIM
D = 16
NP = F // 2
ROW = 256
OFFS = np.concatenate([[0], np.cumsum([FIELD_DIM] * F)[:-1]]).astype(np.int32)

NW = 32
GROUP_B = 4
IDX_G = GROUP_B * F
VCHUNK = 2048


def _tc_build_table(embT):
    grid = (FD + VCHUNK - 1) // VCHUNK

    def rne(bits):
        return bits + 0x7FFF + ((bits >> 16) & 1)

    def body(in_ref, out_ref):
        t = in_ref[...]
        bits = lax.bitcast_convert_type(t, jnp.int32)
        parts = []
        for p in range(NP):
            ha = rne(bits[2 * p])
            hb = rne(bits[2 * p + 1])
            parts.append((hb & (-65536)) | ((ha >> 16) & 0xFFFF))
        inter = jnp.concatenate(parts, axis=0)
        out_ref[:, 0:NP * D] = inter.T

    return pl.pallas_call(
        body,
        grid=(grid,),
        in_specs=[pl.BlockSpec((F, D, VCHUNK), lambda i: (0, 0, i))],
        out_specs=pl.BlockSpec((VCHUNK, ROW), lambda i: (i, 0)),
        out_shape=jax.ShapeDtypeStruct((FD, ROW), jnp.int32),
    )(embT)


def _sc_gather_ffm(tableT, xo1d, batch):
    n_groups = batch // (NW * GROUP_B)
    idx_pt = batch // NW * F
    mesh = plsc.VectorSubcoreMesh(core_axis_name="c", subcore_axis_name="s")

    @functools.partial(
        pl.kernel,
        out_type=jax.ShapeDtypeStruct((batch * D,), jnp.float32),
        mesh=mesh,
        scratch_types=[
            pltpu.VMEM((idx_pt,), jnp.int32),
            pltpu.VMEM((IDX_G, ROW), jnp.int32),
            pltpu.VMEM((IDX_G, ROW), jnp.int32),
            pltpu.VMEM((2 * GROUP_B * D,), jnp.float32),
            pltpu.SemaphoreType.DMA,
            pltpu.SemaphoreType.DMA,
        ],
        compiler_params=pltpu.CompilerParams(
            use_tc_tiling_on_sc=True, needs_layout_passes=False
        ),
    )
    def k(tab, xo_h, out_h, idxv, rows0, rows1, outb, sem0, sem1):
        wid = lax.axis_index("s") * 2 + lax.axis_index("c")
        gbase = wid * n_groups

        pltpu.sync_copy(xo_h.at[pl.ds(wid * idx_pt, idx_pt)], idxv)

        def fire(rows, sem, grp):
            pltpu.make_async_copy(
                tab.at[idxv.at[pl.ds(grp * IDX_G, IDX_G)]], rows, sem
            ).start()

        def drain(rows, sem, grp):
            pltpu.make_async_copy(
                tab.at[idxv.at[pl.ds(grp * IDX_G, IDX_G)]], rows, sem
            ).wait()

        def load_pair(rows, r, p):
            vi = rows[r, pl.ds(D * p, D)]
            lo = plsc.bitcast(vi << 16, jnp.float32)
            hi = plsc.bitcast(vi & (-65536), jnp.float32)
            return lo, hi

        def compute(rows, slot):
            def body_b(bl, c):
                base = bl * F
                acc = jnp.zeros((D,), jnp.float32)
                for pf in range(NP - 1):
                    for pg in range(pf + 1, NP):
                        ga0, ga1 = load_pair(rows, base + 2 * pg, pf)
                        fa0, fa1 = load_pair(rows, base + 2 * pf, pg)
                        gb0, gb1 = load_pair(rows, base + 2 * pg + 1, pf)
                        fb0, fb1 = load_pair(rows, base + 2 * pf + 1, pg)
                        acc = acc + ga0 * fa0
                        acc = acc + ga1 * fb0
                        acc = acc + gb0 * fa1
                        acc = acc + gb1 * fb1
                for p in range(NP):
                    a, _ = load_pair(rows, base + 2 * p + 1, p)
                    _, b = load_pair(rows, base + 2 * p, p)
                    acc = acc + a * b
                outb[pl.ds((slot * GROUP_B + bl) * D, D)] = acc
                return c
            lax.fori_loop(0, GROUP_B, body_b, 0)

        fire(rows0, sem0, 0)

        def step(it, c):
            g0 = 2 * it
            fire(rows1, sem1, g0 + 1)
            drain(rows0, sem0, g0)
            compute(rows0, 0)

            @pl.when(g0 + 2 < n_groups)
            def _():
                fire(rows0, sem0, g0 + 2)

            drain(rows1, sem1, g0 + 1)
            compute(rows1, 1)
            pltpu.sync_copy(
                outb,
                out_h.at[pl.ds((gbase + g0) * GROUP_B * D, 2 * GROUP_B * D)],
            )
            return c

        lax.fori_loop(0, n_groups // 2, step, 0)

    return k(tableT, xo1d)


def _tc_reduce(partials):
    def body(p_ref, o_ref):
        o_ref[...] = jnp.sum(p_ref[...], axis=1)

    return pl.pallas_call(
        body,
        out_shape=jax.ShapeDtypeStruct((partials.shape[0],), jnp.float32),
    )(partials)


def kernel(x, emb):
    batch = x.shape[0]
    offs = jnp.asarray(OFFS, x.dtype)
    xo1d = (x + offs[None, :]).reshape(batch * F)
    embT = jnp.transpose(emb, (0, 2, 1))
    tableT = _tc_build_table(embT)
    partials = _sc_gather_ffm(tableT, xo1d, batch).reshape(batch, D)
    return _tc_reduce(partials)


# speedup vs baseline: 66.0110x; 1.1188x over previous
import functools

import jax
import jax.numpy as jnp
import numpy as np
from jax import lax
from jax.experimental import pallas as pl
from jax.experimental.pallas import tpu as pltpu
from jax.experimental.pallas import tpu_sc as plsc

F = 26
FIELD_DIM = 3846
FD = F * FIELD_DIM
D = 16
NP = F // 2
ROW = 256
OFFS = np.concatenate([[0], np.cumsum([FIELD_DIM] * F)[:-1]]).astype(np.int32)

NW = 32
GROUP_B = 4
IDX_G = GROUP_B * F
VCHUNK = 2048


def _tc_build_table(embT):
    grid = (FD + VCHUNK - 1) // VCHUNK

    def rne(bits):
        return bits + 0x7FFF + ((bits >> 16) & 1)

    def body(in_ref, out_ref):
        t = in_ref[...]
        bits = lax.bitcast_convert_type(t, jnp.int32)
        parts = []
        for p in range(NP):
            ha = rne(bits[2 * p])
            hb = rne(bits[2 * p + 1])
            parts.append((hb & (-65536)) | ((ha >> 16) & 0xFFFF))
        inter = jnp.concatenate(parts, axis=0)
        out_ref[:, 0:NP * D] = inter.T

    return pl.pallas_call(
        body,
        grid=(grid,),
        in_specs=[pl.BlockSpec((F, D, VCHUNK), lambda i: (0, 0, i))],
        out_specs=pl.BlockSpec((VCHUNK, ROW), lambda i: (i, 0)),
        out_shape=jax.ShapeDtypeStruct((FD, ROW), jnp.int32),
    )(embT)


def _sc_gather_ffm(tableT, xo1d, batch):
    n_groups = batch // (NW * GROUP_B)
    idx_pt = batch // NW * F
    mesh = plsc.VectorSubcoreMesh(core_axis_name="c", subcore_axis_name="s")

    @functools.partial(
        pl.kernel,
        out_type=jax.ShapeDtypeStruct((batch * D,), jnp.float32),
        mesh=mesh,
        scratch_types=[
            pltpu.VMEM((idx_pt,), jnp.int32),
            pltpu.VMEM((IDX_G, ROW), jnp.int32),
            pltpu.VMEM((IDX_G, ROW), jnp.int32),
            pltpu.VMEM((2 * GROUP_B * D,), jnp.float32),
            pltpu.SemaphoreType.DMA,
            pltpu.SemaphoreType.DMA,
        ],
        compiler_params=pltpu.CompilerParams(
            use_tc_tiling_on_sc=True, needs_layout_passes=False
        ),
    )
    def k(tab, xo_h, out_h, idxv, rows0, rows1, outb, sem0, sem1):
        wid = lax.axis_index("s") * 2 + lax.axis_index("c")
        gbase = wid * n_groups

        pltpu.sync_copy(xo_h.at[pl.ds(wid * idx_pt, idx_pt)], idxv)

        def fire(rows, sem, grp):
            pltpu.make_async_copy(
                tab.at[idxv.at[pl.ds(grp * IDX_G, IDX_G)]], rows, sem
            ).start()

        def drain(rows, sem, grp):
            pltpu.make_async_copy(
                tab.at[idxv.at[pl.ds(grp * IDX_G, IDX_G)]], rows, sem
            ).wait()

        def load_pair(rows, r, p):
            vi = rows[r, pl.ds(D * p, D)]
            lo = plsc.bitcast(vi << 16, jnp.float32)
            hi = plsc.bitcast(vi & (-65536), jnp.float32)
            return lo, hi

        def compute(rows, slot):
            def body_b(bl, c):
                base = bl * F
                accs = [jnp.zeros((D,), jnp.float32) for _ in range(4)]
                for pf in range(NP - 1):
                    for pg in range(pf + 1, NP):
                        ga0, ga1 = load_pair(rows, base + 2 * pg, pf)
                        fa0, fa1 = load_pair(rows, base + 2 * pf, pg)
                        gb0, gb1 = load_pair(rows, base + 2 * pg + 1, pf)
                        fb0, fb1 = load_pair(rows, base + 2 * pf + 1, pg)
                        accs[0] = accs[0] + ga0 * fa0
                        accs[1] = accs[1] + ga1 * fb0
                        accs[2] = accs[2] + gb0 * fa1
                        accs[3] = accs[3] + gb1 * fb1
                for p in range(NP):
                    a, _ = load_pair(rows, base + 2 * p + 1, p)
                    _, b = load_pair(rows, base + 2 * p, p)
                    accs[p % 4] = accs[p % 4] + a * b
                acc = (accs[0] + accs[1]) + (accs[2] + accs[3])
                outb[pl.ds((slot * GROUP_B + bl) * D, D)] = acc
                return c
            lax.fori_loop(0, GROUP_B, body_b, 0)

        fire(rows0, sem0, 0)

        def step(it, c):
            g0 = 2 * it
            fire(rows1, sem1, g0 + 1)
            drain(rows0, sem0, g0)
            compute(rows0, 0)

            @pl.when(g0 + 2 < n_groups)
            def _():
                fire(rows0, sem0, g0 + 2)

            drain(rows1, sem1, g0 + 1)
            compute(rows1, 1)
            pltpu.sync_copy(
                outb,
                out_h.at[pl.ds((gbase + g0) * GROUP_B * D, 2 * GROUP_B * D)],
            )
            return c

        lax.fori_loop(0, n_groups // 2, step, 0)

    return k(tableT, xo1d)


def _tc_reduce(partials):
    def body(p_ref, o_ref):
        o_ref[...] = jnp.sum(p_ref[...], axis=1)

    return pl.pallas_call(
        body,
        out_shape=jax.ShapeDtypeStruct((partials.shape[0],), jnp.float32),
    )(partials)


def kernel(x, emb):
    batch = x.shape[0]
    offs = jnp.asarray(OFFS, x.dtype)
    xo1d = (x + offs[None, :]).reshape(batch * F)
    embT = jnp.transpose(emb, (0, 2, 1))
    tableT = _tc_build_table(embT)
    partials = _sc_gather_ffm(tableT, xo1d, batch).reshape(batch, D)
    return _tc_reduce(partials)


# speedup vs baseline: 68.7013x; 1.0408x over previous
import functools

import jax
import jax.numpy as jnp
import numpy as np
from jax import lax
from jax.experimental import pallas as pl
from jax.experimental.pallas import tpu as pltpu
from jax.experimental.pallas import tpu_sc as plsc

F = 26
FIELD_DIM = 3846
FD = F * FIELD_DIM
D = 16
NP = F // 2
ROW = 256
OFFS = np.concatenate([[0], np.cumsum([FIELD_DIM] * F)[:-1]]).astype(np.int32)

NW = 32
GROUP_B = 4
IDX_G = GROUP_B * F
VCHUNK = 2048


def _tc_build_table(embT):
    grid = (FD + VCHUNK - 1) // VCHUNK

    def rne(bits):
        return bits + 0x7FFF + ((bits >> 16) & 1)

    def body(in_ref, out_ref):
        t = in_ref[...]
        bits = lax.bitcast_convert_type(t, jnp.int32)
        parts = []
        for p in range(NP):
            ha = rne(bits[2 * p])
            hb = rne(bits[2 * p + 1])
            parts.append((hb & (-65536)) | ((ha >> 16) & 0xFFFF))
        inter = jnp.concatenate(parts, axis=0)
        out_ref[:, 0:NP * D] = inter.T

    return pl.pallas_call(
        body,
        grid=(grid,),
        in_specs=[pl.BlockSpec((F, D, VCHUNK), lambda i: (0, 0, i))],
        out_specs=pl.BlockSpec((VCHUNK, ROW), lambda i: (i, 0)),
        out_shape=jax.ShapeDtypeStruct((FD, ROW), jnp.int32),
    )(embT)


def _sc_gather_ffm(tableT, xo1d, batch):
    n_groups = batch // (NW * GROUP_B)
    idx_pt = batch // NW * F
    mesh = plsc.VectorSubcoreMesh(core_axis_name="c", subcore_axis_name="s")

    @functools.partial(
        pl.kernel,
        out_type=jax.ShapeDtypeStruct((batch,), jnp.float32),
        mesh=mesh,
        scratch_types=[
            pltpu.VMEM((idx_pt,), jnp.int32),
            pltpu.VMEM((IDX_G, ROW), jnp.int32),
            pltpu.VMEM((IDX_G, ROW), jnp.int32),
            pltpu.VMEM((D,), jnp.float32),
            pltpu.SemaphoreType.DMA,
            pltpu.SemaphoreType.DMA,
        ],
        compiler_params=pltpu.CompilerParams(
            use_tc_tiling_on_sc=True, needs_layout_passes=False
        ),
    )
    def k(tab, xo_h, out_h, idxv, rows0, rows1, outv, sem0, sem1):
        wid = lax.axis_index("s") * 2 + lax.axis_index("c")
        gbase = wid * n_groups

        pltpu.sync_copy(xo_h.at[pl.ds(wid * idx_pt, idx_pt)], idxv)

        def fire(rows, sem, grp):
            pltpu.make_async_copy(
                tab.at[idxv.at[pl.ds(grp * IDX_G, IDX_G)]], rows, sem
            ).start()

        def drain(rows, sem, grp):
            pltpu.make_async_copy(
                tab.at[idxv.at[pl.ds(grp * IDX_G, IDX_G)]], rows, sem
            ).wait()

        def load_pair(rows, r, p):
            vi = rows[r, pl.ds(D * p, D)]
            lo = plsc.bitcast(vi << 16, jnp.float32)
            hi = plsc.bitcast(vi & (-65536), jnp.float32)
            return lo, hi

        def compute(rows, slot, base_lane, acc16):
            lanes = lax.iota(jnp.int32, D)

            def body_b(bl, acc16):
                base = bl * F
                accs = [jnp.zeros((D,), jnp.float32) for _ in range(4)]
                for pf in range(NP - 1):
                    for pg in range(pf + 1, NP):
                        ga0, ga1 = load_pair(rows, base + 2 * pg, pf)
                        fa0, fa1 = load_pair(rows, base + 2 * pf, pg)
                        gb0, gb1 = load_pair(rows, base + 2 * pg + 1, pf)
                        fb0, fb1 = load_pair(rows, base + 2 * pf + 1, pg)
                        accs[0] = accs[0] + ga0 * fa0
                        accs[1] = accs[1] + ga1 * fb0
                        accs[2] = accs[2] + gb0 * fa1
                        accs[3] = accs[3] + gb1 * fb1
                for p in range(NP):
                    a, _ = load_pair(rows, base + 2 * p + 1, p)
                    _, b = load_pair(rows, base + 2 * p, p)
                    accs[p % 4] = accs[p % 4] + a * b
                acc = (accs[0] + accs[1]) + (accs[2] + accs[3])
                total = jnp.sum(acc)
                return jnp.where(lanes == base_lane + bl, total, acc16)
            return lax.fori_loop(0, GROUP_B, body_b, acc16)

        fire(rows0, sem0, 0)

        def step(it, acc16):
            g0 = 2 * it
            base_lane = (it % 2) * 2 * GROUP_B
            fire(rows1, sem1, g0 + 1)
            drain(rows0, sem0, g0)
            acc16 = compute(rows0, 0, base_lane, acc16)

            @pl.when(g0 + 2 < n_groups)
            def _():
                fire(rows0, sem0, g0 + 2)

            drain(rows1, sem1, g0 + 1)
            acc16 = compute(rows1, 1, base_lane + GROUP_B, acc16)

            @pl.when(it % 2 == 1)
            def _():
                outv[...] = acc16
                pltpu.sync_copy(
                    outv,
                    out_h.at[pl.ds(wid * (batch // NW) + (it - 1) * 2 * GROUP_B, D)],
                )

            return acc16

        lax.fori_loop(0, n_groups // 2, step, jnp.zeros((D,), jnp.float32))

    return k(tableT, xo1d)


def kernel(x, emb):
    batch = x.shape[0]
    offs = jnp.asarray(OFFS, x.dtype)
    xo1d = (x + offs[None, :]).reshape(batch * F)
    embT = jnp.transpose(emb, (0, 2, 1))
    tableT = _tc_build_table(embT)
    return _sc_gather_ffm(tableT, xo1d, batch)
